# baseline (speedup 1.0000x reference)
import functools

import jax
import jax.numpy as jnp
from jax import lax
from jax.experimental import pallas as pl
from jax.experimental.pallas import tpu as pltpu
from jax.experimental.pallas import tpu_sc as plsc

M = 100000
D = 128
B = 16384

NC = 2
NS = 16
KB = 128

R = 14208
NDUMP = 64
NCHUNK = -(-M // R)
EPT = B // NS
EPW = B // (NC * NS)

_mesh = plsc.VectorSubcoreMesh(core_axis_name="c", subcore_axis_name="s")


@functools.partial(
    pl.kernel,
    out_type=jax.ShapeDtypeStruct((M, D), jnp.float32),
    mesh=_mesh,
    scratch_types=[
        pltpu.VMEM((KB,), jnp.int32),
        pltpu.VMEM((KB,), jnp.int32),
        pltpu.VMEM((KB, D), jnp.float32),
        pltpu.VMEM_SHARED((R + NDUMP, D), jnp.float32),
    ],
)
def _scatter_kernel(mem_hbm, idx_hbm, val_hbm, upd_hbm, ibuf, tgt, valb, acc):
    core = lax.axis_index("c")
    sid = lax.axis_index("s")
    lane = lax.iota(jnp.int32, 16)
    dump_base = R + sid * 4

    for c in range(NCHUNK):
        base = c * R
        rows = min(R, M - base)
        rpt = -(-rows // NS)
        rpt = -(-rpt // 8) * 8

        @pl.when(core == (c % NC))
        def _chunk():
            off = jnp.minimum(sid * rpt, rows - rpt)
            pltpu.sync_copy(mem_hbm.at[pl.ds(base + off, rpt)],
                            acc.at[pl.ds(off, rpt)])
            plsc.subcore_barrier()

            ebase = sid * EPT

            @pl.loop(0, EPT, step=KB)
            def _batch(j):
                pltpu.sync_copy(idx_hbm.at[pl.ds(ebase + j, KB)], ibuf)
                pltpu.sync_copy(val_hbm.at[pl.ds(ebase + j, KB)], valb)

                @pl.loop(0, KB, step=16)
                def _vec(t):
                    v = ibuf[pl.ds(t, 16)]
                    loc = v - base
                    inr = (loc >= 0) & (loc < rows)
                    dump = dump_base + (lane & 3)
                    tgt[pl.ds(t, 16)] = jnp.where(inr, loc, dump)

                pltpu.sync_copy(valb, acc.at[tgt], add=True)

            plsc.subcore_barrier()

            pltpu.sync_copy(acc.at[pl.ds(off, rpt)],
                            upd_hbm.at[pl.ds(base + off, rpt)])
            plsc.subcore_barrier()


@functools.partial(
    pl.kernel,
    out_type=jax.ShapeDtypeStruct((B, D), jnp.float32),
    mesh=_mesh,
    scratch_types=[
        pltpu.VMEM((KB,), jnp.int32),
        pltpu.VMEM((KB, D), jnp.float32),
        pltpu.SemaphoreType.DMA,
    ],
)
def _gather_kernel(upd_hbm, idx_hbm, out_hbm, tgt, rowsb, sem):
    wid = lax.axis_index("s") * NC + lax.axis_index("c")
    base = wid * EPW

    @pl.loop(0, EPW, step=KB)
    def _batch(j):
        pltpu.sync_copy(idx_hbm.at[pl.ds(base + j, KB)], tgt)
        pltpu.async_copy(upd_hbm.at[tgt], rowsb, sem).wait()
        pltpu.sync_copy(rowsb, out_hbm.at[pl.ds(base + j, KB)])


def kernel(mem, idx, val):
    updated = _scatter_kernel(mem, idx, val)
    gathered = _gather_kernel(updated, idx)
    return updated, gathered


# speedup vs baseline: 1.4898x; 1.4898x over previous
import functools

import jax
import jax.numpy as jnp
from jax import lax
from jax.experimental import pallas as pl
from jax.experimental.pallas import tpu as pltpu
from jax.experimental.pallas import tpu_sc as plsc

M = 100000
D = 128
B = 16384

NC = 2
NS = 16
KB = 128

R = 12928
NDUMP = 64
NCHUNK = -(-M // R)
NSLOT = NCHUNK // NC
EPT = B // NS
EPW = B // (NC * NS)
BCAP = EPT + KB

_mesh = plsc.VectorSubcoreMesh(core_axis_name="c", subcore_axis_name="s")
_cp = pltpu.CompilerParams(needs_layout_passes=False)


@functools.partial(
    pl.kernel,
    out_type=jax.ShapeDtypeStruct((M, D), jnp.float32),
    mesh=_mesh,
    scratch_types=[
        pltpu.VMEM((EPT,), jnp.int32),
    ]
    + [pltpu.VMEM((BCAP,), jnp.int32) for _ in range(2 * NSLOT)]
    + [
        pltpu.VMEM((KB,), jnp.int32),
        pltpu.VMEM((KB,), jnp.int32),
        pltpu.VMEM((KB, D), jnp.float32),
        pltpu.VMEM_SHARED((R + NDUMP, D), jnp.float32),
    ],
    compiler_params=_cp,
)
def _scatter_kernel(mem_hbm, idx_hbm, val_hbm, upd_hbm,
                    idxv, *rest):
    btgts = rest[0:NSLOT]
    bsrcs = rest[NSLOT:2 * NSLOT]
    tgt, src, valb, acc = rest[2 * NSLOT:]
    core = lax.axis_index("c")
    sid = lax.axis_index("s")
    lane = lax.iota(jnp.int32, 16)
    dump = R + sid * 4 + (lane & 3)
    ebase = sid * EPT

    pltpu.sync_copy(idx_hbm.at[pl.ds(ebase, EPT)], idxv)

    def _group(g, offs):
        v = idxv[pl.ds(g * 16, 16)]
        gid = ebase + g * 16 + lane
        new = []
        for s in range(NSLOT):
            base = (s * NC + core) * R
            loc = v - base
            m = (loc >= 0) & (loc < R)
            off = offs[s]
            plsc.store_compressed(btgts[s].at[pl.ds(off, 16)], loc, mask=m)
            plsc.store_compressed(bsrcs[s].at[pl.ds(off, 16)], gid, mask=m)
            new.append(off + jnp.sum(m.astype(jnp.int32)))
        return tuple(new)

    cnts = lax.fori_loop(0, EPT // 16, _group, (0,) * NSLOT)

    for s in range(NSLOT):
        for t in range(KB // 16):
            btgts[s][pl.ds(cnts[s] + t * 16, 16)] = dump
            bsrcs[s][pl.ds(cnts[s] + t * 16, 16)] = lane + t * 16

    for c in range(NCHUNK):
        base = c * R
        rows = min(R, M - base)
        rpt = -(-rows // NS)
        rpt = -(-rpt // 8) * 8
        slot = c // NC

        @pl.when(core == (c % NC))
        def _chunk():
            off = jnp.minimum(sid * rpt, rows - rpt)
            pltpu.sync_copy(mem_hbm.at[pl.ds(base + off, rpt)],
                            acc.at[pl.ds(off, rpt)])
            plsc.subcore_barrier()

            trips = (cnts[slot] + KB - 1) >> 7

            def _trip(j, carry):
                for t in range(KB // 16):
                    tgt[pl.ds(t * 16, 16)] = btgts[slot][pl.ds(j * KB + t * 16, 16)]
                    src[pl.ds(t * 16, 16)] = bsrcs[slot][pl.ds(j * KB + t * 16, 16)]
                pltpu.sync_copy(val_hbm.at[src], valb)
                pltpu.sync_copy(valb, acc.at[tgt], add=True)
                return carry

            lax.fori_loop(0, trips, _trip, 0)
            plsc.subcore_barrier()

            pltpu.sync_copy(acc.at[pl.ds(off, rpt)],
                            upd_hbm.at[pl.ds(base + off, rpt)])
            plsc.subcore_barrier()


@functools.partial(
    pl.kernel,
    out_type=jax.ShapeDtypeStruct((B, D), jnp.float32),
    mesh=_mesh,
    scratch_types=[
        pltpu.VMEM((KB,), jnp.int32),
        pltpu.VMEM((KB, D), jnp.float32),
        pltpu.SemaphoreType.DMA,
    ],
)
def _gather_kernel(upd_hbm, idx_hbm, out_hbm, tgt, rowsb, sem):
    wid = lax.axis_index("s") * NC + lax.axis_index("c")
    base = wid * EPW

    @pl.loop(0, EPW, step=KB)
    def _batch(j):
        pltpu.sync_copy(idx_hbm.at[pl.ds(base + j, KB)], tgt)
        pltpu.async_copy(upd_hbm.at[tgt], rowsb, sem).wait()
        pltpu.sync_copy(rowsb, out_hbm.at[pl.ds(base + j, KB)])


def kernel(mem, idx, val):
    updated = _scatter_kernel(mem, idx, val)
    gathered = _gather_kernel(updated, idx)
    return updated, gathered


# speedup vs baseline: 1.7135x; 1.1502x over previous
import functools

import jax
import jax.numpy as jnp
from jax import lax
from jax.experimental import pallas as pl
from jax.experimental.pallas import tpu as pltpu
from jax.experimental.pallas import tpu_sc as plsc

M = 100000
D = 128
B = 16384

NC = 2
NS = 16
KB = 128

R = 2560
RPT = -(-(-(-R // NS)) // 8) * 8
NBUF = 3
NDUMP = 32
NCHUNK = -(-M // R)
NSLOT = NCHUNK // NC
EPT = B // NS
EPW = B // (NC * NS)
BCAP = EPT + 32

_mesh = plsc.VectorSubcoreMesh(core_axis_name="c", subcore_axis_name="s")
_cp = pltpu.CompilerParams(needs_layout_passes=False)


@functools.partial(
    pl.kernel,
    out_type=jax.ShapeDtypeStruct((M, D), jnp.float32),
    mesh=_mesh,
    scratch_types=[
        pltpu.VMEM((EPT,), jnp.int32),
    ]
    + [pltpu.VMEM((BCAP,), jnp.int32) for _ in range(2 * NSLOT)]
    + [
        pltpu.VMEM((KB,), jnp.int32),
        pltpu.VMEM((KB,), jnp.int32),
        pltpu.VMEM((32,), jnp.int32),
        pltpu.VMEM((32,), jnp.int32),
        pltpu.VMEM((KB, D), jnp.float32),
    ]
    + [pltpu.VMEM_SHARED((R + NDUMP, D), jnp.float32) for _ in range(NBUF)]
    + [pltpu.SemaphoreType.DMA for _ in range(2 * NBUF)],
    compiler_params=_cp,
)
def _scatter_kernel(mem_hbm, idx_hbm, val_hbm, upd_hbm, idxv, *rest):
    btgts = rest[0:NSLOT]
    bsrcs = rest[NSLOT:2 * NSLOT]
    tgt, src, tgt32, src32, valb = rest[2 * NSLOT:2 * NSLOT + 5]
    bufs = rest[2 * NSLOT + 5:2 * NSLOT + 5 + NBUF]
    sin = rest[2 * NSLOT + 5 + NBUF:2 * NSLOT + 5 + 2 * NBUF]
    sout = rest[2 * NSLOT + 5 + 2 * NBUF:]
    core = lax.axis_index("c")
    sid = lax.axis_index("s")
    lane = lax.iota(jnp.int32, 16)
    dump = R + sid * 2 + (lane & 1)
    ebase = sid * EPT

    pltpu.sync_copy(idx_hbm.at[pl.ds(ebase, EPT)], idxv)

    def _group(g, offs):
        v = idxv[pl.ds(g * 16, 16)]
        gid = ebase + g * 16 + lane
        new = []
        for s in range(NSLOT):
            base = (s * NC + core) * R
            loc = v - base
            m = (loc >= 0) & (loc < R)
            off = offs[s]
            plsc.store_compressed(btgts[s].at[pl.ds(off, 16)], loc, mask=m)
            plsc.store_compressed(bsrcs[s].at[pl.ds(off, 16)], gid, mask=m)
            new.append(off + jnp.sum(m.astype(jnp.int32)))
        return tuple(new)

    cnts = lax.fori_loop(0, EPT // 16, _group, (0,) * NSLOT)

    for s in range(NSLOT):
        for t in range(2):
            btgts[s][pl.ds(cnts[s] + t * 16, 16)] = dump
            bsrcs[s][pl.ds(cnts[s] + t * 16, 16)] = \
                ebase + (s * 32 + t * 16) % EPT + lane

    def _tile_off(i):
        cb = (i * NC + core) * R
        hi = jnp.minimum(cb + R, M) - RPT - cb
        return cb, jnp.minimum(sid * RPT, hi)

    def _copy_in(i):
        cb, off = _tile_off(i)
        return pltpu.async_copy(mem_hbm.at[pl.ds(cb + off, RPT)],
                                bufs[i % NBUF].at[pl.ds(off, RPT)],
                                sin[i % NBUF])

    def _writeback(i):
        cb, off = _tile_off(i)
        return pltpu.async_copy(bufs[i % NBUF].at[pl.ds(off, RPT)],
                                upd_hbm.at[pl.ds(cb + off, RPT)],
                                sout[i % NBUF])

    def _scatter(i):
        buf = bufs[i % NBUF]
        trips = cnts[i] >> 7
        full = trips << 7
        trips32 = (cnts[i] - full + 31) >> 5

        def _trip(j, carry):
            for t in range(KB // 16):
                tgt[pl.ds(t * 16, 16)] = btgts[i][pl.ds(j * KB + t * 16, 16)]
                src[pl.ds(t * 16, 16)] = bsrcs[i][pl.ds(j * KB + t * 16, 16)]
            pltpu.sync_copy(val_hbm.at[src], valb)
            pltpu.sync_copy(valb, buf.at[tgt], add=True)
            return carry

        def _trip32(j, carry):
            for t in range(2):
                tgt32[pl.ds(t * 16, 16)] = \
                    btgts[i][pl.ds(full + j * 32 + t * 16, 16)]
                src32[pl.ds(t * 16, 16)] = \
                    bsrcs[i][pl.ds(full + j * 32 + t * 16, 16)]
            pltpu.sync_copy(val_hbm.at[src32], valb.at[pl.ds(0, 32)])
            pltpu.sync_copy(valb.at[pl.ds(0, 32)], buf.at[tgt32], add=True)
            return carry

        lax.fori_loop(0, trips, _trip, 0)
        lax.fori_loop(0, trips32, _trip32, 0)

    h_in = {0: _copy_in(0), 1: _copy_in(1)}
    h_out = {}
    for i in range(NSLOT):
        h_in.pop(i).wait()
        plsc.subcore_barrier()
        if i >= 1:
            h_out[i - 1] = _writeback(i - 1)
        _scatter(i)
        if i + 2 < NSLOT:
            if i + 2 - NBUF >= 0:
                h_out.pop(i + 2 - NBUF).wait()
            h_in[i + 2] = _copy_in(i + 2)
    plsc.subcore_barrier()
    h_out[NSLOT - 1] = _writeback(NSLOT - 1)
    for i in sorted(h_out):
        h_out[i].wait()


@functools.partial(
    pl.kernel,
    out_type=jax.ShapeDtypeStruct((B, D), jnp.float32),
    mesh=_mesh,
    scratch_types=[
        pltpu.VMEM((KB,), jnp.int32),
        pltpu.VMEM((KB, D), jnp.float32),
        pltpu.SemaphoreType.DMA,
    ],
)
def _gather_kernel(upd_hbm, idx_hbm, out_hbm, tgt, rowsb, sem):
    wid = lax.axis_index("s") * NC + lax.axis_index("c")
    base = wid * EPW

    @pl.loop(0, EPW, step=KB)
    def _batch(j):
        pltpu.sync_copy(idx_hbm.at[pl.ds(base + j, KB)], tgt)
        pltpu.async_copy(upd_hbm.at[tgt], rowsb, sem).wait()
        pltpu.sync_copy(rowsb, out_hbm.at[pl.ds(base + j, KB)])


def kernel(mem, idx, val):
    updated = _scatter_kernel(mem, idx, val)
    gathered = _gather_kernel(updated, idx)
    return updated, gathered


# speedup vs baseline: 1.7509x; 1.0219x over previous
import functools

import jax
import jax.numpy as jnp
from jax import lax
from jax.experimental import pallas as pl
from jax.experimental.pallas import tpu as pltpu
from jax.experimental.pallas import tpu_sc as plsc

M = 100000
D = 128
B = 16384

NC = 2
NS = 16
KB = 128

R = 2816
RPT = R // NS
NBUF = 3
NDUMP = 32
GPAD = 32
NCHUNK = -(-M // R)
NSLOT = NCHUNK // NC
NR = NSLOT // NBUF
EPT = B // NS
BCAP = EPT + 32

_mesh = plsc.VectorSubcoreMesh(core_axis_name="c", subcore_axis_name="s")
_cp = pltpu.CompilerParams(needs_layout_passes=False)


@functools.partial(
    pl.kernel,
    out_type=[jax.ShapeDtypeStruct((M, D), jnp.float32),
              jax.ShapeDtypeStruct((B + GPAD, D), jnp.float32)],
    mesh=_mesh,
    scratch_types=[
        pltpu.VMEM((EPT,), jnp.int32),
        pltpu.VMEM((NSLOT * BCAP,), jnp.int32),
        pltpu.VMEM((NSLOT * BCAP,), jnp.int32),
        pltpu.VMEM((32,), jnp.int32),
        pltpu.VMEM((KB,), jnp.int32),
        pltpu.VMEM((KB,), jnp.int32),
        pltpu.VMEM((32,), jnp.int32),
        pltpu.VMEM((32,), jnp.int32),
        pltpu.VMEM((KB, D), jnp.float32),
    ]
    + [pltpu.VMEM_SHARED((R + NDUMP, D), jnp.float32) for _ in range(NBUF)]
    + [pltpu.SemaphoreType.DMA for _ in range(2 * NBUF)],
    compiler_params=_cp,
)
def _ground_kernel(mem_hbm, idx_hbm, val_hbm, upd_hbm, gat_hbm,
                   idxv, btgt, bsrc, cnt_arr, tgt, src, tgt32, src32, valb,
                   *rest):
    bufs = rest[0:NBUF]
    sin = rest[NBUF:2 * NBUF]
    sout = rest[2 * NBUF:]
    core = lax.axis_index("c")
    sid = lax.axis_index("s")
    lane = lax.iota(jnp.int32, 16)
    dump = R + sid * 2 + (lane & 1)
    ebase = sid * EPT

    pltpu.sync_copy(idx_hbm.at[pl.ds(ebase, EPT)], idxv)

    def _group(g, offs):
        v = idxv[pl.ds(g * 16, 16)]
        gid = ebase + g * 16 + lane
        new = []
        for s in range(NSLOT):
            base = (s * NC + core) * R
            loc = v - base
            m = (loc >= 0) & (loc < R)
            off = offs[s]
            plsc.store_compressed(btgt.at[pl.ds(s * BCAP + off, 16)],
                                  loc, mask=m)
            plsc.store_compressed(bsrc.at[pl.ds(s * BCAP + off, 16)],
                                  gid, mask=m)
            new.append(off + jnp.sum(m.astype(jnp.int32)))
        return tuple(new)

    cnts = lax.fori_loop(0, EPT // 16, _group, (0,) * NSLOT)

    for s in range(NSLOT):
        for t in range(2):
            btgt[pl.ds(s * BCAP + cnts[s] + t * 16, 16)] = dump
            bsrc[pl.ds(s * BCAP + cnts[s] + t * 16, 16)] = B + t * 16 + lane

    for h in range(2):
        acc = lane * 0
        for k in range(16):
            if h * 16 + k < NSLOT:
                acc = jnp.where(lane == k, cnts[h * 16 + k], acc)
        cnt_arr[pl.ds(h * 16, 16)] = acc

    def _cnt(slot):
        vec = jnp.where(slot < 16, cnt_arr[pl.ds(0, 16)],
                        cnt_arr[pl.ds(16, 16)])
        return jnp.sum(jnp.where(lane == (slot & 15), vec, 0))

    def _tile_off(slot):
        cb = (slot * NC + core) * R
        hi = jnp.minimum(cb + R, M) - RPT - cb
        return cb, jnp.minimum(sid * RPT, hi)

    def _issue_in(slot, b):
        cb, off = _tile_off(slot)
        pltpu.async_copy(mem_hbm.at[pl.ds(cb + off, RPT)],
                         bufs[b].at[pl.ds(off, RPT)], sin[b])

    def _wait_in(b):
        pltpu.make_async_copy(mem_hbm.at[pl.ds(0, RPT)],
                              bufs[b].at[pl.ds(0, RPT)], sin[b]).wait()

    def _issue_out(slot, b):
        cb, off = _tile_off(slot)
        pltpu.async_copy(bufs[b].at[pl.ds(off, RPT)],
                         upd_hbm.at[pl.ds(cb + off, RPT)], sout[b])

    def _wait_out(b):
        pltpu.make_async_copy(bufs[b].at[pl.ds(0, RPT)],
                              upd_hbm.at[pl.ds(0, RPT)], sout[b]).wait()

    def _scatter(slot, b):
        buf = bufs[b]
        n = _cnt(slot)
        trips = n >> 7
        full = trips << 7
        trips32 = (n - full + 31) >> 5
        bbase = slot * BCAP

        def _trip(j, carry):
            for t in range(KB // 16):
                tgt[pl.ds(t * 16, 16)] = \
                    btgt[pl.ds(bbase + j * KB + t * 16, 16)]
                src[pl.ds(t * 16, 16)] = \
                    bsrc[pl.ds(bbase + j * KB + t * 16, 16)]
            pltpu.sync_copy(val_hbm.at[src], valb)
            pltpu.sync_copy(valb, buf.at[tgt], add=True)
            return carry

        def _trip32(j, carry):
            for t in range(2):
                pos = full + j * 32 + t * 16
                keep = (pos + lane) < n
                safe = ebase + ((slot * 32 + t * 16) & (EPT - 1)) + lane
                tgt32[pl.ds(t * 16, 16)] = btgt[pl.ds(bbase + pos, 16)]
                src32[pl.ds(t * 16, 16)] = \
                    jnp.where(keep, bsrc[pl.ds(bbase + pos, 16)], safe)
            pltpu.sync_copy(val_hbm.at[src32], valb.at[pl.ds(0, 32)])
            pltpu.sync_copy(valb.at[pl.ds(0, 32)], buf.at[tgt32], add=True)
            return carry

        lax.fori_loop(0, trips, _trip, 0)
        lax.fori_loop(0, trips32, _trip32, 0)

    def _collect(slot, b):
        buf = bufs[b]
        n = _cnt(slot)
        trips = n >> 7
        full = trips << 7
        trips32 = (n - full + 31) >> 5
        bbase = slot * BCAP

        def _trip(j, carry):
            for t in range(KB // 16):
                tgt[pl.ds(t * 16, 16)] = \
                    btgt[pl.ds(bbase + j * KB + t * 16, 16)]
                src[pl.ds(t * 16, 16)] = \
                    bsrc[pl.ds(bbase + j * KB + t * 16, 16)]
            pltpu.sync_copy(buf.at[tgt], valb)
            pltpu.sync_copy(valb, gat_hbm.at[src])
            return carry

        def _trip32(j, carry):
            for t in range(2):
                pos = full + j * 32 + t * 16
                tgt32[pl.ds(t * 16, 16)] = btgt[pl.ds(bbase + pos, 16)]
                src32[pl.ds(t * 16, 16)] = bsrc[pl.ds(bbase + pos, 16)]
            pltpu.sync_copy(buf.at[tgt32], valb.at[pl.ds(0, 32)])
            pltpu.sync_copy(valb.at[pl.ds(0, 32)], gat_hbm.at[src32])
            return carry

        lax.fori_loop(0, trips, _trip, 0)
        lax.fori_loop(0, trips32, _trip32, 0)

    _issue_in(0, 0)
    _issue_in(1, 1)

    def _round(r, carry):
        for b in range(NBUF):
            slot = r * NBUF + b
            prev = slot - 1
            b_prev = (b - 1) % NBUF
            _wait_in(b)
            plsc.subcore_barrier()
            if b == 0:
                @pl.when(r >= 1)
                def _prev_work():
                    _issue_out(prev, b_prev)
                    _collect(prev, b_prev)
            else:
                _issue_out(prev, b_prev)
                _collect(prev, b_prev)
            _scatter(slot, b)
            if b == 0:
                @pl.when(r >= 1)
                def _wait_prev():
                    _wait_out(b_prev)
                _issue_in(slot + 2, b_prev)
            else:
                _wait_out(b_prev)

                @pl.when(r < NR - 1)
                def _ahead():
                    _issue_in(slot + 2, b_prev)
        return carry

    lax.fori_loop(0, NR, _round, 0)
    plsc.subcore_barrier()
    last = NSLOT - 1
    bl = last % NBUF
    _issue_out(last, bl)
    _collect(last, bl)
    _wait_out(bl)


def kernel(mem, idx, val):
    updated, gathered_pad = _ground_kernel(mem, idx, val)
    return updated, gathered_pad[:B]
